# baseline (speedup 1.0000x reference)
import functools

import jax
import jax.numpy as jnp
from jax import lax
from jax.experimental import pallas as pl
from jax.experimental.pallas import tpu as pltpu
from jax.experimental.pallas import tpu_sc as plsc

VOCAB = 100000
D = 64
NLANE = 16
NVREG = D // NLANE
B = 16384
L = 200
NCLS = 5

NC, NS = 2, 16
NW = NC * NS
RPW = B // NW
CH = 16
NCHUNK = RPW // CH
RPG = 1
GPC = CH // RPG
GL = RPG * L
CHL = CH * L
NBUF = 8
AHEAD = NBUF - 1


def _body(x_hbm, table_hbm, w_hbm, b_hbm, out_hbm,
          ibuf, gbuf, out_v, wb_v, semi, semg):
    cid = lax.axis_index("c")
    sid = lax.axis_index("s")
    wid = sid * NC + cid
    row0 = wid * RPW

    iota = lax.iota(jnp.int32, NLANE)

    def idx_start(ci, n):
        pltpu.make_async_copy(
            x_hbm.at[pl.ds(row0 + ci * CH, CH)], ibuf.at[n], semi.at[n]).start()

    def idx_wait(n):
        pltpu.make_async_copy(
            x_hbm.at[pl.ds(0, CH)], ibuf.at[n], semi.at[n]).wait()

    def gather_start(k, p, q):
        for rr in range(RPG):
            row = RPG * k + rr
            pltpu.make_async_copy(
                table_hbm.at[ibuf.at[p, row, pl.ds(0, 128)]],
                gbuf.at[q, pl.ds(rr * L, 128)], semg.at[q]).start()
            pltpu.make_async_copy(
                table_hbm.at[ibuf.at[p, row, pl.ds(128, L - 128)]],
                gbuf.at[q, pl.ds(rr * L + 128, L - 128)], semg.at[q]).start()

    def gather_wait(q):
        pltpu.make_async_copy(
            table_hbm.at[pl.ds(0, GL)], gbuf.at[q], semg.at[q]).wait()

    pltpu.sync_copy(w_hbm, wb_v.at[pl.ds(0, NCLS)])
    pltpu.sync_copy(b_hbm, wb_v.at[NCLS])
    wvec = [[wb_v[c, pl.ds(v * NLANE, NLANE)] for v in range(NVREG)]
            for c in range(NCLS)]
    bvec = wb_v[NCLS, pl.ds(0, NLANE)]

    def reduce_group(q, ci, k):
        for r in range(RPG):
            base = r * L

            def body(i, accs, _base=base, _q=q):
                return tuple(
                    jnp.maximum(a, gbuf[_q, _base + i, pl.ds(c * NLANE, NLANE)])
                    for c, a in enumerate(accs))

            init = tuple(
                jnp.full((NLANE,), -jnp.inf, jnp.float32) for _ in range(NVREG))
            accs = lax.fori_loop(0, L, body, init, unroll=10)
            orow = bvec
            for c in range(NCLS):
                t = accs[0] * wvec[c][0]
                for v in range(1, NVREG):
                    t = t + accs[v] * wvec[c][v]
                e = jnp.sum(t)
                orow = orow + jnp.where(iota == c, e, 0.0)
            out_v[ci * CH + k * RPG + r] = orow

    pltpu.sync_copy(x_hbm.at[pl.ds(row0, CH)], ibuf.at[0])
    idx_start(1, 1)
    for k in range(AHEAD):
        gather_start(k, 0, k % NBUF)

    def pair_body(cp, _):
        for offc in (0, 1):
            ci = cp * 2 + offc
            p = offc
            for k in range(GPC):
                q = k % NBUF
                gather_wait(q)
                kt = k + AHEAD
                qt = kt % NBUF
                if kt < GPC:
                    gather_start(kt, p, qt)
                elif kt == GPC:
                    @pl.when(ci + 1 < NCHUNK)
                    def _():
                        idx_wait(1 - p)
                        gather_start(0, 1 - p, qt)
                else:
                    @pl.when(ci + 1 < NCHUNK)
                    def _():
                        gather_start(kt - GPC, 1 - p, qt)
                if k == GPC - 1:
                    @pl.when(ci + 2 < NCHUNK)
                    def _():
                        idx_start(ci + 2, p)
                reduce_group(q, ci, k)
        return 0

    lax.fori_loop(0, NCHUNK // 2, pair_body, 0)
    pltpu.sync_copy(out_v, out_hbm.at[pl.ds(row0, RPW)])


_run = functools.partial(
    pl.kernel,
    out_type=jax.ShapeDtypeStruct((B, NLANE), jnp.float32),
    mesh=plsc.VectorSubcoreMesh(core_axis_name="c", subcore_axis_name="s"),
    scratch_types=[
        pltpu.VMEM((2, CH, L), jnp.int32),
        pltpu.VMEM((NBUF, GL, D), jnp.float32),
        pltpu.VMEM((RPW, NLANE), jnp.float32),
        pltpu.VMEM((NCLS + 1, D), jnp.float32),
        pltpu.SemaphoreType.DMA((2,)),
        pltpu.SemaphoreType.DMA((NBUF,)),
    ],
    compiler_params=pltpu.CompilerParams(use_tc_tiling_on_sc=False, needs_layout_passes=False),
)(_body)


def kernel(x, emb_table, fc_w, fc_b):
    b_pad = jnp.zeros((D,), jnp.float32).at[:NCLS].set(fc_b)
    out = _run(x.astype(jnp.int32), emb_table, fc_w, b_pad)
    return out[:, :NCLS]


# speedup vs baseline: 1.3974x; 1.3974x over previous
import functools

import jax
import jax.numpy as jnp
from jax import lax
from jax.experimental import pallas as pl
from jax.experimental.pallas import tpu as pltpu
from jax.experimental.pallas import tpu_sc as plsc

VOCAB = 100000
D = 64
NLANE = 16
NVREG = D // NLANE
B = 16384
L = 200
NCLS = 5

NC, NS = 2, 16
NW = NC * NS
RPW = B // NW
CH = 16
NCHUNK = RPW // CH
RPG = 1
GPC = CH // RPG
GL = RPG * L
CHL = CH * L
NBUF = 8
AHEAD = NBUF - 1


def _body(x_hbm, table_hbm, w_hbm, b_hbm, out_hbm,
          ibuf, gbuf, out_v, wb_v, semi, semg):
    cid = lax.axis_index("c")
    sid = lax.axis_index("s")
    wid = sid * NC + cid
    row0 = wid * RPW

    iota = lax.iota(jnp.int32, NLANE)

    def idx_start(ci, n):
        pltpu.make_async_copy(
            x_hbm.at[pl.ds(row0 + ci * CH, CH)], ibuf.at[n], semi.at[n]).start()

    def idx_wait(n):
        pltpu.make_async_copy(
            x_hbm.at[pl.ds(0, CH)], ibuf.at[n], semi.at[n]).wait()

    def gather_start(k, p, q):
        for rr in range(RPG):
            row = RPG * k + rr
            pltpu.make_async_copy(
                table_hbm.at[ibuf.at[p, row, pl.ds(0, 128)]],
                gbuf.at[q, pl.ds(rr * L, 128)], semg.at[q]).start()
            pltpu.make_async_copy(
                table_hbm.at[ibuf.at[p, row, pl.ds(128, L - 128)]],
                gbuf.at[q, pl.ds(rr * L + 128, L - 128)], semg.at[q]).start()

    def gather_wait(q):
        pltpu.make_async_copy(
            table_hbm.at[pl.ds(0, GL)], gbuf.at[q], semg.at[q]).wait()

    pltpu.sync_copy(w_hbm, wb_v.at[pl.ds(0, NCLS)])
    pltpu.sync_copy(b_hbm, wb_v.at[NCLS])
    wvec = [[wb_v[c, pl.ds(v * NLANE, NLANE)] for v in range(NVREG)]
            for c in range(NCLS)]
    bvec = wb_v[NCLS, pl.ds(0, NLANE)]

    def reduce_group(q, ci, k):
        for r in range(RPG):
            base = r * L

            def body(i, accs, _base=base, _q=q):
                return tuple(
                    jnp.maximum(a, gbuf[_q, _base + i, pl.ds(c * 2 * NLANE,
                                                             2 * NLANE)])
                    for c, a in enumerate(accs))

            init = tuple(
                jnp.full((2 * NLANE,), -jnp.inf, jnp.bfloat16)
                for _ in range(NVREG // 2))
            accs16 = lax.fori_loop(0, L, body, init, unroll=10)
            accs = []
            for a16 in accs16:
                ea, eb = plsc.unpack(a16, format=plsc.PackFormat.INTERLEAVED)
                accs.extend((ea, eb))
            orow = bvec
            for c in range(NCLS):
                t = accs[0] * wvec[c][0]
                for v in range(1, NVREG):
                    t = t + accs[v] * wvec[c][v]
                e = jnp.sum(t)
                orow = orow + jnp.where(iota == c, e, 0.0)
            out_v[ci * CH + k * RPG + r] = orow

    pltpu.sync_copy(x_hbm.at[pl.ds(row0, CH)], ibuf.at[0])
    idx_start(1, 1)
    for k in range(AHEAD):
        gather_start(k, 0, k % NBUF)

    def pair_body(cp, _):
        for offc in (0, 1):
            ci = cp * 2 + offc
            p = offc
            for k in range(GPC):
                q = k % NBUF
                gather_wait(q)
                kt = k + AHEAD
                qt = kt % NBUF
                if kt < GPC:
                    gather_start(kt, p, qt)
                elif kt == GPC:
                    @pl.when(ci + 1 < NCHUNK)
                    def _():
                        idx_wait(1 - p)
                        gather_start(0, 1 - p, qt)
                else:
                    @pl.when(ci + 1 < NCHUNK)
                    def _():
                        gather_start(kt - GPC, 1 - p, qt)
                if k == GPC - 1:
                    @pl.when(ci + 2 < NCHUNK)
                    def _():
                        idx_start(ci + 2, p)
                reduce_group(q, ci, k)
        return 0

    lax.fori_loop(0, NCHUNK // 2, pair_body, 0)
    pltpu.sync_copy(out_v, out_hbm.at[pl.ds(row0, RPW)])


_run = functools.partial(
    pl.kernel,
    out_type=jax.ShapeDtypeStruct((B, NLANE), jnp.float32),
    mesh=plsc.VectorSubcoreMesh(core_axis_name="c", subcore_axis_name="s"),
    scratch_types=[
        pltpu.VMEM((2, CH, L), jnp.int32),
        pltpu.VMEM((NBUF, GL, D), jnp.bfloat16),
        pltpu.VMEM((RPW, NLANE), jnp.float32),
        pltpu.VMEM((NCLS + 1, D), jnp.float32),
        pltpu.SemaphoreType.DMA((2,)),
        pltpu.SemaphoreType.DMA((NBUF,)),
    ],
    compiler_params=pltpu.CompilerParams(use_tc_tiling_on_sc=False, needs_layout_passes=False),
)(_body)


_PERM = [d for h in range(2) for par in range(2)
         for d in range(32 * h + par, 32 * h + 32, 2)]


def kernel(x, emb_table, fc_w, fc_b):
    b_pad = jnp.zeros((D,), jnp.float32).at[:NCLS].set(fc_b)
    out = _run(x.astype(jnp.int32), emb_table.astype(jnp.bfloat16),
               fc_w[:, jnp.array(_PERM)], b_pad)
    return out[:, :NCLS]
